# baseline (speedup 1.0000x reference)
import functools

import jax
import jax.numpy as jnp
from jax import lax
from jax.experimental import pallas as pl
from jax.experimental.pallas import tpu as pltpu
from jax.experimental.pallas import tpu_sc as plsc

_B, _S, _D = 4, 4096, 1024
_NC, _NS = 2, 16
_NW = _NC * _NS
_SEQ_PER_W = _S // _NW
_C = 16
_NCHUNK = _SEQ_PER_W // _C
_NSTEP = _NCHUNK * _B
_NB = 5
_SUB = 4
_CS = _C // _SUB


def _sc_add(inputs, embeddings):
    mesh = plsc.VectorSubcoreMesh(core_axis_name="c", subcore_axis_name="s")

    @functools.partial(
        pl.kernel,
        mesh=mesh,
        out_type=jax.ShapeDtypeStruct((_B, _S, _D), jnp.float32),
        scratch_types=[
            pltpu.VMEM((_C, _D), jnp.float32),
            pltpu.VMEM((_C, _D), jnp.float32),
            pltpu.VMEM((_C, _D), jnp.float32),
            pltpu.VMEM((_C, _D), jnp.float32),
            pltpu.VMEM((_C, _D), jnp.float32),
            pltpu.VMEM((_C, _D), jnp.float32),
            pltpu.VMEM((_C, _D), jnp.float32),
            pltpu.SemaphoreType.DMA,
            pltpu.SemaphoreType.DMA,
            pltpu.SemaphoreType.DMA,
            pltpu.SemaphoreType.DMA,
            pltpu.SemaphoreType.DMA,
            pltpu.SemaphoreType.DMA,
            pltpu.SemaphoreType.DMA,
            pltpu.SemaphoreType.DMA,
            pltpu.SemaphoreType.DMA,
            pltpu.SemaphoreType.DMA,
            pltpu.SemaphoreType.DMA,
            pltpu.SemaphoreType.DMA,
        ],
    )
    def k(in_hbm, emb_hbm, out_hbm, ib0, ib1, ib2, ib3, ib4, eb0, eb1,
          sin0, sin1, sin2, sin3, sin4, sout0, sout1, sout2, sout3, sout4,
          se0, se1):
        wid = lax.axis_index("s") * _NC + lax.axis_index("c")
        seq0 = wid * _SEQ_PER_W
        ibufs = (ib0, ib1, ib2, ib3, ib4)
        ebufs = (eb0, eb1)
        sins = (sin0, sin1, sin2, sin3, sin4)
        souts = (sout0, sout1, sout2, sout3, sout4)
        ses = (se0, se1)

        def row0(g):
            i = g // _B
            return pl.multiple_of(seq0 + i * _C, _C)

        def start_in(g):
            p, b = g % _NB, g % _B
            pltpu.make_async_copy(
                in_hbm.at[b, pl.ds(row0(g), _C)], ibufs[p], sins[p]
            ).start()

        def wait_in(g):
            p, b = g % _NB, g % _B
            pltpu.make_async_copy(
                in_hbm.at[b, pl.ds(row0(g), _C)], ibufs[p], sins[p]
            ).wait()

        def start_out_sub(g, u):
            p, b = g % _NB, g % _B
            r = pl.multiple_of(row0(g) + u * _CS, _CS)
            pltpu.make_async_copy(
                ibufs[p].at[pl.ds(u * _CS, _CS)],
                out_hbm.at[b, pl.ds(r, _CS)],
                souts[p],
            ).start()

        def wait_out(g):
            p, b = g % _NB, g % _B
            for u in range(_SUB):
                r = pl.multiple_of(row0(g) + u * _CS, _CS)
                pltpu.make_async_copy(
                    ibufs[p].at[pl.ds(u * _CS, _CS)],
                    out_hbm.at[b, pl.ds(r, _CS)],
                    souts[p],
                ).wait()

        def emb_copy(i):
            off = pl.multiple_of(seq0 + i * _C, _C)
            q = i % 2
            return pltpu.make_async_copy(
                emb_hbm.at[pl.ds(off, _C)], ebufs[q], ses[q]
            )

        def add_sub(p, q, u):
            @plsc.parallel_loop(0, _D, step=16)
            def _(col):
                for r in range(u * _CS, (u + 1) * _CS):
                    plsc.addupdate(
                        ibufs[p].at[r, pl.ds(col, 16)],
                        ebufs[q][r, pl.ds(col, 16)],
                    )

        emb_copy(0).start()
        for g0 in range(_NB - 1):
            start_in(g0)

        for g in range(_NSTEP):
            i, b = divmod(g, _B)
            if b == 0:
                emb_copy(i).wait()
                if i + 1 < _NCHUNK:
                    emb_copy(i + 1).start()
            wait_in(g)
            for u in range(_SUB):
                add_sub(g % _NB, i % 2, u)
                start_out_sub(g, u)
            if g + _NB - 1 < _NSTEP:
                if g >= 1:
                    wait_out(g - 1)
                start_in(g + _NB - 1)

        for g in range(_NSTEP - _NB, _NSTEP):
            wait_out(g)

    return k(inputs, embeddings)


def kernel(inputs, embeddings):
    return _sc_add(inputs, embeddings)


# speedup vs baseline: 1.0399x; 1.0399x over previous
import functools

import jax
import jax.numpy as jnp
from jax import lax
from jax.experimental import pallas as pl
from jax.experimental.pallas import tpu as pltpu
from jax.experimental.pallas import tpu_sc as plsc

_B, _S, _D = 4, 4096, 1024
_NC, _NS = 2, 16
_NW = _NC * _NS
_SEQ_PER_W = _S // _NW
_C = 16
_NCHUNK = _SEQ_PER_W // _C
_NSTEP = _NCHUNK * _B
_NB = 5
_SUB = 4
_CS = _C // _SUB


def _sc_add(inputs, embeddings):
    mesh = plsc.VectorSubcoreMesh(core_axis_name="c", subcore_axis_name="s")

    @functools.partial(
        pl.kernel,
        mesh=mesh,
        out_type=jax.ShapeDtypeStruct((_B, _S, _D), jnp.float32),
        scratch_types=[
            pltpu.VMEM((_C, _D), jnp.float32),
            pltpu.VMEM((_C, _D), jnp.float32),
            pltpu.VMEM((_C, _D), jnp.float32),
            pltpu.VMEM((_C, _D), jnp.float32),
            pltpu.VMEM((_C, _D), jnp.float32),
            pltpu.VMEM((_C, _D), jnp.float32),
            pltpu.VMEM((_C, _D), jnp.float32),
            pltpu.SemaphoreType.DMA,
            pltpu.SemaphoreType.DMA,
            pltpu.SemaphoreType.DMA,
            pltpu.SemaphoreType.DMA,
            pltpu.SemaphoreType.DMA,
            pltpu.SemaphoreType.DMA,
            pltpu.SemaphoreType.DMA,
            pltpu.SemaphoreType.DMA,
            pltpu.SemaphoreType.DMA,
            pltpu.SemaphoreType.DMA,
            pltpu.SemaphoreType.DMA,
            pltpu.SemaphoreType.DMA,
        ],
    )
    def k(in_hbm, emb_hbm, out_hbm, ib0, ib1, ib2, ib3, ib4, eb0, eb1,
          sin0, sin1, sin2, sin3, sin4, sout0, sout1, sout2, sout3, sout4,
          se0, se1):
        wid = lax.axis_index("s") * _NC + lax.axis_index("c")
        seq0 = wid * _SEQ_PER_W
        ibufs = (ib0, ib1, ib2, ib3, ib4)
        ebufs = (eb0, eb1)
        sins = (sin0, sin1, sin2, sin3, sin4)
        souts = (sout0, sout1, sout2, sout3, sout4)
        ses = (se0, se1)

        def row0(g):
            i = g // _B
            return pl.multiple_of(seq0 + i * _C, _C)

        def start_in(g):
            p, b = g % _NB, g % _B
            pltpu.make_async_copy(
                in_hbm.at[b, pl.ds(row0(g), _C)], ibufs[p], sins[p]
            ).start()

        def wait_in(g):
            p, b = g % _NB, g % _B
            pltpu.make_async_copy(
                in_hbm.at[b, pl.ds(row0(g), _C)], ibufs[p], sins[p]
            ).wait()

        def start_out(g):
            p, b = g % _NB, g % _B
            pltpu.make_async_copy(
                ibufs[p], out_hbm.at[b, pl.ds(row0(g), _C)], souts[p]
            ).start()

        def wait_out(g):
            p, b = g % _NB, g % _B
            pltpu.make_async_copy(
                ibufs[p], out_hbm.at[b, pl.ds(row0(g), _C)], souts[p]
            ).wait()

        def emb_copy(i):
            off = pl.multiple_of(seq0 + i * _C, _C)
            q = i % 2
            return pltpu.make_async_copy(
                emb_hbm.at[pl.ds(off, _C)], ebufs[q], ses[q]
            )

        def add_into(p, q):
            @plsc.parallel_loop(0, _D, step=16)
            def _(col):
                for r in range(_C):
                    plsc.addupdate(
                        ibufs[p].at[r, pl.ds(col, 16)],
                        ebufs[q][r, pl.ds(col, 16)],
                    )

        emb_copy(0).start()
        for g0 in range(_NB - 1):
            start_in(g0)

        for g in range(_NSTEP):
            i, b = divmod(g, _B)
            if b == 0:
                emb_copy(i).wait()
                if i + 1 < _NCHUNK:
                    emb_copy(i + 1).start()
            wait_in(g)
            add_into(g % _NB, i % 2)
            start_out(g)
            if g + _NB - 1 < _NSTEP:
                if g >= 1:
                    wait_out(g - 1)
                start_in(g + _NB - 1)

        for g in range(_NSTEP - _NB, _NSTEP):
            wait_out(g)

    return k(inputs, embeddings)


def kernel(inputs, embeddings):
    return _sc_add(inputs, embeddings)
